# baseline (speedup 1.0000x reference)
import functools

import jax
import jax.numpy as jnp
from jax import lax
from jax.experimental import pallas as pl
from jax.experimental.pallas import tpu as pltpu
from jax.experimental.pallas import tpu_sc as plsc

N = 50000
E = 800000
D = 64
G = 256

NPAD = 50176
EPAD = 802816
B = 128
CHUNKS = 392
TILE_E = CHUNKS * B
RPT = NPAD // 16
BLK = 512
F32 = jnp.float32


@functools.partial(
    pl.kernel,
    out_type=jax.ShapeDtypeStruct((2 * NPAD, 32), F32),
    mesh=plsc.VectorSubcoreMesh(core_axis_name="c", subcore_axis_name="s"),
    compiler_params=pltpu.CompilerParams(use_tc_tiling_on_sc=False),
    scratch_types=[
        pltpu.VMEM((B,), jnp.int32),
        pltpu.VMEM((B,), jnp.int32),
        pltpu.VMEM((B,), jnp.int32),
        pltpu.VMEM((B, 64), F32),
        pltpu.VMEM((B, 64), F32),
        pltpu.VMEM((B, 64), F32),
        pltpu.VMEM((B, 32), F32),
        pltpu.VMEM_SHARED((NPAD, 32), F32),
        pltpu.SemaphoreType.DMA,
        pltpu.SemaphoreType.DMA,
    ],
)
def _sc_edge(ts_hbm, td_hbm, ce_hbm, gs_hbm, gd_hbm, ss_hbm, zer_hbm,
             out_hbm, gsi, gdi, ssi, pa, pb, cc, mm, agg, sem1, sem2):
    c = lax.axis_index("c")
    s = lax.axis_index("s")
    pltpu.sync_copy(zer_hbm, agg.at[pl.ds(s * RPT, RPT)])
    plsc.subcore_barrier()

    ebase = s * TILE_E

    def chunk(k, carry):
        off = ebase + k * B
        goff = c * EPAD + off
        pltpu.sync_copy(gs_hbm.at[pl.ds(goff, B)], gsi)
        pltpu.sync_copy(gd_hbm.at[pl.ds(goff, B)], gdi)
        pltpu.sync_copy(ss_hbm.at[pl.ds(off, B)], ssi)
        pltpu.sync_copy(ce_hbm.at[pl.ds(goff, B)], cc)
        cp1 = pltpu.async_copy(ts_hbm.at[gsi], pa, sem1)
        cp2 = pltpu.async_copy(td_hbm.at[gdi], pb, sem2)
        cp1.wait()
        cp2.wait()

        def edge(e, carry2):
            for j in range(2):
                fo = j * 16
                so = 32 + j * 16
                f = pa[e, pl.ds(fo, 16)] + pb[e, pl.ds(fo, 16)] + cc[e, pl.ds(fo, 16)]
                g = pa[e, pl.ds(so, 16)] + pb[e, pl.ds(so, 16)] + cc[e, pl.ds(so, 16)]
                sig = 1.0 / (1.0 + jnp.exp(-f))
                t = jnp.exp(-jnp.abs(g))
                u = t / (2.0 + t)
                u2 = u * u
                lp = 2.0 * u * (1.0 + u2 * (0.33333334 + u2 * (0.2 + u2 * 0.14285715)))
                sp = jnp.maximum(g, 0.0) + lp
                mm[e, pl.ds(fo, 16)] = sig * sp
            return carry2

        lax.fori_loop(0, B, edge, 0)
        pltpu.sync_copy(mm, agg.at[ssi], add=True)
        return carry

    lax.fori_loop(0, CHUNKS, chunk, 0)
    plsc.subcore_barrier()
    pltpu.sync_copy(agg.at[pl.ds(s * RPT, RPT)],
                    out_hbm.at[pl.ds(c * NPAD + s * RPT, RPT)])


def _proj_write(xx, wa_ref, wb_ref, ts_ref, td_ref):
    ta = jnp.dot(xx, wa_ref[...], preferred_element_type=F32)
    ts_ref[0] = ta[:, :64]
    ts_ref[1] = ta[:, 64:]
    tb = jnp.dot(xx, wb_ref[...], preferred_element_type=F32)
    td_ref[0] = tb[:, :64]
    td_ref[1] = tb[:, 64:]


def _fc1_body(x_ref, wfc_ref, bfc_ref, wa_ref, wb_ref, xx_ref, ts_ref, td_ref):
    xx = jnp.maximum(jnp.dot(x_ref[...], wfc_ref[...],
                             preferred_element_type=F32) + bfc_ref[...], 0.0)
    xx_ref[...] = xx
    _proj_write(xx, wa_ref, wb_ref, ts_ref, td_ref)


def _upd_body(xp_ref, a0_ref, a1_ref, inv_ref, wa_ref, wb_ref,
              y_ref, ts_ref, td_ref):
    agg = jnp.concatenate([a0_ref[...], a1_ref[...]], axis=1)
    y = jnp.maximum(xp_ref[...] + agg * inv_ref[...], 0.0)
    y_ref[...] = y
    _proj_write(y, wa_ref, wb_ref, ts_ref, td_ref)


def _gru_body(xp_ref, a0_ref, a1_ref, inv_ref, h_ref, wih_ref, whh_ref,
              bih_ref, bhh_ref, wa_ref, wb_ref, h_out_ref, ts_ref, td_ref):
    agg = jnp.concatenate([a0_ref[...], a1_ref[...]], axis=1)
    m = jnp.maximum(xp_ref[...] + agg * inv_ref[...], 0.0)
    h = h_ref[...]
    gi = jnp.dot(m, wih_ref[...], preferred_element_type=F32) + bih_ref[...]
    gh = jnp.dot(h, whh_ref[...], preferred_element_type=F32) + bhh_ref[...]
    r = jax.nn.sigmoid(gi[:, :64] + gh[:, :64])
    z = jax.nn.sigmoid(gi[:, 64:128] + gh[:, 64:128])
    n = jnp.tanh(gi[:, 128:] + r * gh[:, 128:])
    hn = (1.0 - z) * n + z * h
    h_out_ref[...] = hn
    _proj_write(hn, wa_ref, wb_ref, ts_ref, td_ref)


def _econst_body(ea_ref, wfe_ref, wse_ref, bf_ref, bs_ref, out_ref):
    cf = jnp.dot(ea_ref[...], wfe_ref[...], preferred_element_type=F32) + bf_ref[...]
    cs = jnp.dot(ea_ref[...], wse_ref[...], preferred_element_type=F32) + bs_ref[...]
    out_ref[0] = jnp.concatenate([cf[:, :32], cs[:, :32]], axis=1)
    out_ref[1] = jnp.concatenate([cf[:, 32:], cs[:, 32:]], axis=1)


def _readout_body(xc_ref, wsh_ref, bsh_ref, w2c_ref, b2c_ref, w3c_ref,
                  b3c_ref, w2d_ref, b2d_ref, w3d_ref, b3d_ref, oc_ref, od_ref):
    h = jnp.maximum(jnp.dot(xc_ref[...], wsh_ref[...],
                            preferred_element_type=F32) + bsh_ref[...], 0.0)
    a = jnp.maximum(jnp.dot(h, w2c_ref[...],
                            preferred_element_type=F32) + b2c_ref[...], 0.0)
    oc_ref[...] = jax.nn.sigmoid(
        jnp.dot(a, w3c_ref[...], preferred_element_type=F32) + b3c_ref[...])
    d = jnp.maximum(jnp.dot(h, w2d_ref[...],
                            preferred_element_type=F32) + b2d_ref[...], 0.0)
    od_ref[...] = jnp.dot(d, w3d_ref[...], preferred_element_type=F32) + b3d_ref[...]


_GRID = (NPAD // BLK,)


def _full(shape):
    return pl.BlockSpec(shape, lambda i: tuple(0 for _ in shape))


_NODE_OUT = [
    jax.ShapeDtypeStruct((NPAD, 64), F32),
    jax.ShapeDtypeStruct((2, NPAD, 64), F32),
    jax.ShapeDtypeStruct((2, NPAD, 64), F32),
]
_NODE_OUT_SPECS = [
    pl.BlockSpec((BLK, 64), lambda i: (i, 0)),
    pl.BlockSpec((2, BLK, 64), lambda i: (0, i, 0)),
    pl.BlockSpec((2, BLK, 64), lambda i: (0, i, 0)),
]

_fc1_proj = pl.pallas_call(
    _fc1_body,
    grid=_GRID,
    in_specs=[
        pl.BlockSpec((BLK, 9), lambda i: (i, 0)),
        _full((9, 64)), _full((1, 64)), _full((64, 128)), _full((64, 128)),
    ],
    out_specs=_NODE_OUT_SPECS,
    out_shape=_NODE_OUT,
)

_upd_proj = pl.pallas_call(
    _upd_body,
    grid=_GRID,
    in_specs=[
        pl.BlockSpec((BLK, 64), lambda i: (i, 0)),
        pl.BlockSpec((BLK, 32), lambda i: (i, 0)),
        pl.BlockSpec((BLK, 32), lambda i: (i, 0)),
        pl.BlockSpec((BLK, 1), lambda i: (i, 0)),
        _full((64, 128)), _full((64, 128)),
    ],
    out_specs=_NODE_OUT_SPECS,
    out_shape=_NODE_OUT,
)

_gru_proj = pl.pallas_call(
    _gru_body,
    grid=_GRID,
    in_specs=[
        pl.BlockSpec((BLK, 64), lambda i: (i, 0)),
        pl.BlockSpec((BLK, 32), lambda i: (i, 0)),
        pl.BlockSpec((BLK, 32), lambda i: (i, 0)),
        pl.BlockSpec((BLK, 1), lambda i: (i, 0)),
        pl.BlockSpec((BLK, 64), lambda i: (i, 0)),
        _full((64, 192)), _full((64, 192)), _full((1, 192)), _full((1, 192)),
        _full((64, 128)), _full((64, 128)),
    ],
    out_specs=_NODE_OUT_SPECS,
    out_shape=_NODE_OUT,
)

_edge_const = pl.pallas_call(
    _econst_body,
    grid=(EPAD // BLK,),
    in_specs=[
        pl.BlockSpec((BLK, 3), lambda i: (i, 0)),
        _full((3, 64)), _full((3, 64)), _full((1, 64)), _full((1, 64)),
    ],
    out_specs=[pl.BlockSpec((2, BLK, 64), lambda i: (0, i, 0))],
    out_shape=[jax.ShapeDtypeStruct((2, EPAD, 64), F32)],
)

_readout = pl.pallas_call(
    _readout_body,
    out_shape=[jax.ShapeDtypeStruct((G, 2), F32),
               jax.ShapeDtypeStruct((G, 2), F32)],
)


def kernel(x, edge_index_1, weight_1, edge_index_2, weight_2, batch,
           W_fc1, b_fc1, Wf, bf, Ws, bs, Wih, Whh, bih, bhh,
           W_sh, b_sh, W2c, b2c, W3c, b3c, W2d, b2d, W3d, b3d):
    WfA, WfB, WfE = Wf[:64], Wf[64:128], Wf[128:]
    WsA, WsB, WsE = Ws[:64], Ws[64:128], Ws[128:]
    WA = jnp.concatenate([WfA[:, :32], WsA[:, :32], WfA[:, 32:], WsA[:, 32:]], axis=1)
    WB = jnp.concatenate([WfB[:, :32], WsB[:, :32], WfB[:, 32:], WsB[:, 32:]], axis=1)
    bf2 = bf.reshape(1, 64)
    bs2 = bs.reshape(1, 64)

    xp = jnp.zeros((NPAD, 9), F32).at[:N].set(x)
    zer = jnp.zeros((RPT, 32), F32)

    def prep_edges(ei, ea):
        src = ei[0].astype(jnp.int32)
        dst = ei[1].astype(jnp.int32)
        padv = jnp.full((EPAD - E,), N, jnp.int32)
        srcp = jnp.concatenate([src, padv])
        dstp = jnp.concatenate([dst, padv])
        gs2 = jnp.concatenate([srcp, srcp + NPAD])
        gd2 = jnp.concatenate([dstp, dstp + NPAD])
        cnt = jnp.bincount(src, length=NPAD).astype(F32)
        inv = (1.0 / jnp.maximum(cnt, 1.0)).reshape(NPAD, 1)
        eap = jnp.zeros((EPAD, 3), F32).at[:E].set(ea)
        (ce,) = _edge_const(eap, WfE, WsE, bf2, bs2)
        return gs2, gd2, srcp, inv, ce.reshape(2 * EPAD, 64)

    gs2_1, gd2_1, ss_1, inv1, C1 = prep_edges(edge_index_1, weight_1)
    gs2_2, gd2_2, ss_2, inv2, C2 = prep_edges(edge_index_2, weight_2)

    xx, ts, td = _fc1_proj(xp, W_fc1, b_fc1.reshape(1, 64), WA, WB)
    h = xx
    WihT = Wih.T
    WhhT = Whh.T
    bih2 = bih.reshape(1, 192)
    bhh2 = bhh.reshape(1, 192)

    for _ in range(3):
        agg = _sc_edge(ts.reshape(2 * NPAD, 64), td.reshape(2 * NPAD, 64),
                       C2, gs2_2, gd2_2, ss_2, zer).reshape(2, NPAD, 32)
        y, ts, td = _upd_proj(xx, agg[0], agg[1], inv2, WA, WB)
        agg = _sc_edge(ts.reshape(2 * NPAD, 64), td.reshape(2 * NPAD, 64),
                       C1, gs2_1, gd2_1, ss_1, zer).reshape(2, NPAD, 32)
        h, ts, td = _gru_proj(y, agg[0], agg[1], inv1, h, WihT, WhhT,
                              bih2, bhh2, WA, WB)
        xx = h

    uniq = jnp.unique(batch, size=G)
    idx_ct = jnp.searchsorted(batch, uniq)
    xc = xx[idx_ct]
    oc, od = _readout(xc, W_sh, b_sh.reshape(1, 64), W2c, b2c.reshape(1, 64),
                      W3c, b3c.reshape(1, 2), W2d, b2d.reshape(1, 64),
                      W3d, b3d.reshape(1, 2))
    return (oc, od)


# speedup vs baseline: 1.1494x; 1.1494x over previous
import functools

import jax
import jax.numpy as jnp
from jax import lax
from jax.experimental import pallas as pl
from jax.experimental.pallas import tpu as pltpu
from jax.experimental.pallas import tpu_sc as plsc

N = 50000
E = 800000
D = 64
G = 256

NPAD = 50176
EPAD = 802816
B = 64
CHUNKS = 784
TILE_E = CHUNKS * B
RPT = NPAD // 16
BLK = 512
F32 = jnp.float32


NCHK = 16 * CHUNKS
NITER = CHUNKS // 2


@functools.partial(
    pl.kernel,
    out_type=jax.ShapeDtypeStruct((2 * NPAD, 32), F32),
    mesh=plsc.VectorSubcoreMesh(core_axis_name="c", subcore_axis_name="s"),
    compiler_params=pltpu.CompilerParams(use_tc_tiling_on_sc=False),
    scratch_types=[
        pltpu.VMEM((2, B), jnp.int32),
        pltpu.VMEM((2, B), jnp.int32),
        pltpu.VMEM((1, B), jnp.int32),
        pltpu.VMEM((1, B), jnp.int32),
        pltpu.VMEM((B, 64), F32),
        pltpu.VMEM((B, 64), F32),
        pltpu.VMEM((B, 64), F32),
        pltpu.VMEM((B, 64), F32),
        pltpu.VMEM((B, 64), F32),
        pltpu.VMEM((B, 64), F32),
        pltpu.VMEM((B, 32), F32),
        pltpu.VMEM((B, 32), F32),
        pltpu.VMEM_SHARED((NPAD, 32), F32),
        pltpu.SemaphoreType.DMA,
        pltpu.SemaphoreType.DMA,
        pltpu.SemaphoreType.DMA,
        pltpu.SemaphoreType.DMA,
        pltpu.SemaphoreType.DMA,
        pltpu.SemaphoreType.DMA,
    ],
)
def _sc_edge(ts_hbm, td_hbm, ce_hbm, idx_hbm, zer_hbm, out_hbm,
             gA, gB, sA, sB, paA, paB, pbA, pbB, ccA, ccB, mmA, mmB,
             agg, semLA, semLB, semGA, semGB, semSA, semSB):
    c = lax.axis_index("c")
    s = lax.axis_index("s")
    pltpu.sync_copy(zer_hbm, agg.at[pl.ds(s * RPT, RPT)])
    plsc.subcore_barrier()

    cbase = s * CHUNKS

    def fire_load(m, gbuf, cbuf, sem):
        pltpu.async_copy(idx_hbm.at[pl.ds(2 * (c * NCHK + m), 2)], gbuf, sem)
        pltpu.async_copy(ce_hbm.at[pl.ds(c * EPAD + m * B, B)], cbuf, sem)

    def drain_load(gbuf, cbuf, sem):
        pltpu.make_async_copy(idx_hbm.at[pl.ds(0, 2)], gbuf, sem).wait()
        pltpu.make_async_copy(ce_hbm.at[pl.ds(0, B)], cbuf, sem).wait()

    def fire_gather(gbuf, pa, pb, sem):
        pltpu.async_copy(ts_hbm.at[gbuf.at[0]], pa, sem)
        pltpu.async_copy(td_hbm.at[gbuf.at[1]], pb, sem)

    def drain_gather(gbuf, pa, pb, sem):
        pltpu.make_async_copy(ts_hbm.at[gbuf.at[0]], pa, sem).wait()
        pltpu.make_async_copy(td_hbm.at[gbuf.at[1]], pb, sem).wait()

    def fire_scatter(mm, sbuf, sem):
        pltpu.async_copy(mm, agg.at[sbuf.at[0]], sem, add=True)

    def drain_scatter(mm, sbuf, sem):
        pltpu.make_async_copy(mm, agg.at[sbuf.at[0]], sem).wait()

    def compute(gbuf, sbuf, pa, pb, cc, mm):
        cnp = c * NPAD
        for j in range(B // 16):
            sbuf[0, pl.ds(j * 16, 16)] = gbuf[0, pl.ds(j * 16, 16)] - cnp

        def edge(e, carry2):
            for j in range(2):
                fo = j * 16
                so = 32 + j * 16
                f = pa[e, pl.ds(fo, 16)] + pb[e, pl.ds(fo, 16)] + cc[e, pl.ds(fo, 16)]
                g = pa[e, pl.ds(so, 16)] + pb[e, pl.ds(so, 16)] + cc[e, pl.ds(so, 16)]
                sig = 1.0 / (1.0 + jnp.exp(-f))
                t = jnp.exp(-jnp.abs(g))
                u = t / (2.0 + t)
                u2 = u * u
                lp = 2.0 * u * (1.0 + u2 * (0.33333334 + u2 * (0.2 + u2 * (0.14285715 + u2 * 0.11111111))))
                sp = jnp.maximum(g, 0.0) + lp
                mm[e, pl.ds(fo, 16)] = sig * sp
            return carry2

        lax.fori_loop(0, B, edge, 0)

    fire_load(cbase, gA, ccA, semLA)
    fire_load(cbase + 1, gB, ccB, semLB)
    drain_load(gA, ccA, semLA)
    fire_gather(gA, paA, pbA, semGA)

    def body(g, carry):
        m_a = cbase + 2 * g
        drain_gather(gA, paA, pbA, semGA)
        drain_load(gB, ccB, semLB)
        fire_gather(gB, paB, pbB, semGB)

        @pl.when(g > 0)
        def _():
            drain_scatter(mmA, sA, semSA)

        compute(gA, sA, paA, pbA, ccA, mmA)
        fire_scatter(mmA, sA, semSA)

        @pl.when(g < NITER - 1)
        def _():
            fire_load(m_a + 2, gA, ccA, semLA)

        drain_gather(gB, paB, pbB, semGB)

        @pl.when(g > 0)
        def _():
            drain_scatter(mmB, sB, semSB)

        compute(gB, sB, paB, pbB, ccB, mmB)
        fire_scatter(mmB, sB, semSB)

        @pl.when(g < NITER - 1)
        def _():
            fire_load(m_a + 3, gB, ccB, semLB)
            drain_load(gA, ccA, semLA)
            fire_gather(gA, paA, pbA, semGA)

        return carry

    lax.fori_loop(0, NITER, body, 0)
    drain_scatter(mmA, sA, semSA)
    drain_scatter(mmB, sB, semSB)
    plsc.subcore_barrier()
    pltpu.sync_copy(agg.at[pl.ds(s * RPT, RPT)],
                    out_hbm.at[pl.ds(c * NPAD + s * RPT, RPT)])


def _proj_write(xx, wa_ref, wb_ref, ts_ref, td_ref):
    ta = jnp.dot(xx, wa_ref[...], preferred_element_type=F32)
    ts_ref[0] = ta[:, :64]
    ts_ref[1] = ta[:, 64:]
    tb = jnp.dot(xx, wb_ref[...], preferred_element_type=F32)
    td_ref[0] = tb[:, :64]
    td_ref[1] = tb[:, 64:]


def _fc1_body(x_ref, wfc_ref, bfc_ref, wa_ref, wb_ref, xx_ref, ts_ref, td_ref):
    xx = jnp.maximum(jnp.dot(x_ref[...], wfc_ref[...],
                             preferred_element_type=F32) + bfc_ref[...], 0.0)
    xx_ref[...] = xx
    _proj_write(xx, wa_ref, wb_ref, ts_ref, td_ref)


def _upd_body(xp_ref, a0_ref, a1_ref, inv_ref, wa_ref, wb_ref,
              y_ref, ts_ref, td_ref):
    agg = jnp.concatenate([a0_ref[...], a1_ref[...]], axis=1)
    y = jnp.maximum(xp_ref[...] + agg * inv_ref[...], 0.0)
    y_ref[...] = y
    _proj_write(y, wa_ref, wb_ref, ts_ref, td_ref)


def _gru_body(xp_ref, a0_ref, a1_ref, inv_ref, h_ref, wih_ref, whh_ref,
              bih_ref, bhh_ref, wa_ref, wb_ref, h_out_ref, ts_ref, td_ref):
    agg = jnp.concatenate([a0_ref[...], a1_ref[...]], axis=1)
    m = jnp.maximum(xp_ref[...] + agg * inv_ref[...], 0.0)
    h = h_ref[...]
    gi = jnp.dot(m, wih_ref[...], preferred_element_type=F32) + bih_ref[...]
    gh = jnp.dot(h, whh_ref[...], preferred_element_type=F32) + bhh_ref[...]
    r = jax.nn.sigmoid(gi[:, :64] + gh[:, :64])
    z = jax.nn.sigmoid(gi[:, 64:128] + gh[:, 64:128])
    n = jnp.tanh(gi[:, 128:] + r * gh[:, 128:])
    hn = (1.0 - z) * n + z * h
    h_out_ref[...] = hn
    _proj_write(hn, wa_ref, wb_ref, ts_ref, td_ref)


def _econst_body(ea_ref, wfe_ref, wse_ref, bf_ref, bs_ref, out_ref):
    cf = jnp.dot(ea_ref[...], wfe_ref[...], preferred_element_type=F32) + bf_ref[...]
    cs = jnp.dot(ea_ref[...], wse_ref[...], preferred_element_type=F32) + bs_ref[...]
    out_ref[0] = jnp.concatenate([cf[:, :32], cs[:, :32]], axis=1)
    out_ref[1] = jnp.concatenate([cf[:, 32:], cs[:, 32:]], axis=1)


def _readout_body(xc_ref, wsh_ref, bsh_ref, w2c_ref, b2c_ref, w3c_ref,
                  b3c_ref, w2d_ref, b2d_ref, w3d_ref, b3d_ref, oc_ref, od_ref):
    h = jnp.maximum(jnp.dot(xc_ref[...], wsh_ref[...],
                            preferred_element_type=F32) + bsh_ref[...], 0.0)
    a = jnp.maximum(jnp.dot(h, w2c_ref[...],
                            preferred_element_type=F32) + b2c_ref[...], 0.0)
    oc_ref[...] = jax.nn.sigmoid(
        jnp.dot(a, w3c_ref[...], preferred_element_type=F32) + b3c_ref[...])
    d = jnp.maximum(jnp.dot(h, w2d_ref[...],
                            preferred_element_type=F32) + b2d_ref[...], 0.0)
    od_ref[...] = jnp.dot(d, w3d_ref[...], preferred_element_type=F32) + b3d_ref[...]


_GRID = (NPAD // BLK,)


def _full(shape):
    return pl.BlockSpec(shape, lambda i: tuple(0 for _ in shape))


_NODE_OUT = [
    jax.ShapeDtypeStruct((NPAD, 64), F32),
    jax.ShapeDtypeStruct((2, NPAD, 64), F32),
    jax.ShapeDtypeStruct((2, NPAD, 64), F32),
]
_NODE_OUT_SPECS = [
    pl.BlockSpec((BLK, 64), lambda i: (i, 0)),
    pl.BlockSpec((2, BLK, 64), lambda i: (0, i, 0)),
    pl.BlockSpec((2, BLK, 64), lambda i: (0, i, 0)),
]

_fc1_proj = pl.pallas_call(
    _fc1_body,
    grid=_GRID,
    in_specs=[
        pl.BlockSpec((BLK, 9), lambda i: (i, 0)),
        _full((9, 64)), _full((1, 64)), _full((64, 128)), _full((64, 128)),
    ],
    out_specs=_NODE_OUT_SPECS,
    out_shape=_NODE_OUT,
)

_upd_proj = pl.pallas_call(
    _upd_body,
    grid=_GRID,
    in_specs=[
        pl.BlockSpec((BLK, 64), lambda i: (i, 0)),
        pl.BlockSpec((BLK, 32), lambda i: (i, 0)),
        pl.BlockSpec((BLK, 32), lambda i: (i, 0)),
        pl.BlockSpec((BLK, 1), lambda i: (i, 0)),
        _full((64, 128)), _full((64, 128)),
    ],
    out_specs=_NODE_OUT_SPECS,
    out_shape=_NODE_OUT,
)

_gru_proj = pl.pallas_call(
    _gru_body,
    grid=_GRID,
    in_specs=[
        pl.BlockSpec((BLK, 64), lambda i: (i, 0)),
        pl.BlockSpec((BLK, 32), lambda i: (i, 0)),
        pl.BlockSpec((BLK, 32), lambda i: (i, 0)),
        pl.BlockSpec((BLK, 1), lambda i: (i, 0)),
        pl.BlockSpec((BLK, 64), lambda i: (i, 0)),
        _full((64, 192)), _full((64, 192)), _full((1, 192)), _full((1, 192)),
        _full((64, 128)), _full((64, 128)),
    ],
    out_specs=_NODE_OUT_SPECS,
    out_shape=_NODE_OUT,
)

_edge_const = pl.pallas_call(
    _econst_body,
    grid=(EPAD // BLK,),
    in_specs=[
        pl.BlockSpec((BLK, 3), lambda i: (i, 0)),
        _full((3, 64)), _full((3, 64)), _full((1, 64)), _full((1, 64)),
    ],
    out_specs=[pl.BlockSpec((2, BLK, 64), lambda i: (0, i, 0))],
    out_shape=[jax.ShapeDtypeStruct((2, EPAD, 64), F32)],
)

_readout = pl.pallas_call(
    _readout_body,
    out_shape=[jax.ShapeDtypeStruct((G, 2), F32),
               jax.ShapeDtypeStruct((G, 2), F32)],
)


def kernel(x, edge_index_1, weight_1, edge_index_2, weight_2, batch,
           W_fc1, b_fc1, Wf, bf, Ws, bs, Wih, Whh, bih, bhh,
           W_sh, b_sh, W2c, b2c, W3c, b3c, W2d, b2d, W3d, b3d):
    WfA, WfB, WfE = Wf[:64], Wf[64:128], Wf[128:]
    WsA, WsB, WsE = Ws[:64], Ws[64:128], Ws[128:]
    WA = jnp.concatenate([WfA[:, :32], WsA[:, :32], WfA[:, 32:], WsA[:, 32:]], axis=1)
    WB = jnp.concatenate([WfB[:, :32], WsB[:, :32], WfB[:, 32:], WsB[:, 32:]], axis=1)
    bf2 = bf.reshape(1, 64)
    bs2 = bs.reshape(1, 64)

    xp = jnp.zeros((NPAD, 9), F32).at[:N].set(x)
    zer = jnp.zeros((RPT, 32), F32)

    def prep_edges(ei, ea):
        src = ei[0].astype(jnp.int32)
        dst = ei[1].astype(jnp.int32)
        padv = jnp.full((EPAD - E,), N, jnp.int32)
        srcp = jnp.concatenate([src, padv])
        dstp = jnp.concatenate([dst, padv])
        gs2 = jnp.concatenate([srcp, srcp + NPAD]).reshape(2, NCHK, B)
        gd2 = jnp.concatenate([dstp, dstp + NPAD]).reshape(2, NCHK, B)
        idxr = jnp.stack([gs2, gd2], axis=2).reshape(2 * NCHK * 2, B)
        cnt = jnp.bincount(src, length=NPAD).astype(F32)
        inv = (1.0 / jnp.maximum(cnt, 1.0)).reshape(NPAD, 1)
        eap = jnp.zeros((EPAD, 3), F32).at[:E].set(ea)
        (ce,) = _edge_const(eap, WfE, WsE, bf2, bs2)
        return idxr, inv, ce.reshape(2 * EPAD, 64)

    idx1, inv1, C1 = prep_edges(edge_index_1, weight_1)
    idx2, inv2, C2 = prep_edges(edge_index_2, weight_2)

    xx, ts, td = _fc1_proj(xp, W_fc1, b_fc1.reshape(1, 64), WA, WB)
    h = xx
    WihT = Wih.T
    WhhT = Whh.T
    bih2 = bih.reshape(1, 192)
    bhh2 = bhh.reshape(1, 192)

    for _ in range(3):
        agg = _sc_edge(ts.reshape(2 * NPAD, 64), td.reshape(2 * NPAD, 64),
                       C2, idx2, zer).reshape(2, NPAD, 32)
        y, ts, td = _upd_proj(xx, agg[0], agg[1], inv2, WA, WB)
        agg = _sc_edge(ts.reshape(2 * NPAD, 64), td.reshape(2 * NPAD, 64),
                       C1, idx1, zer).reshape(2, NPAD, 32)
        h, ts, td = _gru_proj(y, agg[0], agg[1], inv1, h, WihT, WhhT,
                              bih2, bhh2, WA, WB)
        xx = h

    uniq = jnp.unique(batch, size=G)
    idx_ct = jnp.searchsorted(batch, uniq)
    xc = xx[idx_ct]
    oc, od = _readout(xc, W_sh, b_sh.reshape(1, 64), W2c, b2c.reshape(1, 64),
                      W3c, b3c.reshape(1, 2), W2d, b2d.reshape(1, 64),
                      W3d, b3d.reshape(1, 2))
    return (oc, od)


# speedup vs baseline: 1.3062x; 1.1364x over previous
import functools

import jax
import jax.numpy as jnp
from jax import lax
from jax.experimental import pallas as pl
from jax.experimental.pallas import tpu as pltpu
from jax.experimental.pallas import tpu_sc as plsc

N = 50000
E = 800000
D = 64
G = 256

NPAD = 50176
EPAD = 802816
B = 64
CHUNKS = 784
TILE_E = CHUNKS * B
RPT = NPAD // 16
BLK = 512
F32 = jnp.float32


NCHK = 16 * CHUNKS
NITER = CHUNKS // 2


@functools.partial(
    pl.kernel,
    out_type=jax.ShapeDtypeStruct((2 * NPAD, 32), F32),
    mesh=plsc.VectorSubcoreMesh(core_axis_name="c", subcore_axis_name="s"),
    compiler_params=pltpu.CompilerParams(use_tc_tiling_on_sc=False),
    scratch_types=[
        pltpu.VMEM((2, B), jnp.int32),
        pltpu.VMEM((2, B), jnp.int32),
        pltpu.VMEM((1, B), jnp.int32),
        pltpu.VMEM((1, B), jnp.int32),
        pltpu.VMEM((B, 64), F32),
        pltpu.VMEM((B, 64), F32),
        pltpu.VMEM((B, 64), F32),
        pltpu.VMEM((B, 64), F32),
        pltpu.VMEM((B, 64), F32),
        pltpu.VMEM((B, 64), F32),
        pltpu.VMEM((B, 32), F32),
        pltpu.VMEM((B, 32), F32),
        pltpu.VMEM_SHARED((NPAD, 32), F32),
        pltpu.SemaphoreType.DMA,
        pltpu.SemaphoreType.DMA,
        pltpu.SemaphoreType.DMA,
        pltpu.SemaphoreType.DMA,
        pltpu.SemaphoreType.DMA,
        pltpu.SemaphoreType.DMA,
    ],
)
def _sc_edge(ts_hbm, td_hbm, ce_hbm, idx_hbm, zer_hbm, out_hbm,
             gA, gB, sA, sB, paA, paB, pbA, pbB, ccA, ccB, mmA, mmB,
             agg, semLA, semLB, semGA, semGB, semSA, semSB):
    c = lax.axis_index("c")
    s = lax.axis_index("s")
    pltpu.sync_copy(zer_hbm, agg.at[pl.ds(s * RPT, RPT)])
    plsc.subcore_barrier()

    cbase = s * CHUNKS

    def fire_load(m, gbuf, cbuf, sem):
        pltpu.async_copy(idx_hbm.at[pl.ds(2 * (c * NCHK + m), 2)], gbuf, sem)
        pltpu.async_copy(ce_hbm.at[pl.ds(c * EPAD + m * B, B)], cbuf, sem)

    def drain_load(gbuf, cbuf, sem):
        pltpu.make_async_copy(idx_hbm.at[pl.ds(0, 2)], gbuf, sem).wait()
        pltpu.make_async_copy(ce_hbm.at[pl.ds(0, B)], cbuf, sem).wait()

    def fire_gather(gbuf, pa, pb, sem):
        pltpu.async_copy(ts_hbm.at[gbuf.at[0]], pa, sem)
        pltpu.async_copy(td_hbm.at[gbuf.at[1]], pb, sem)

    def drain_gather(gbuf, pa, pb, sem):
        pltpu.make_async_copy(ts_hbm.at[gbuf.at[0]], pa, sem).wait()
        pltpu.make_async_copy(td_hbm.at[gbuf.at[1]], pb, sem).wait()

    def fire_scatter(mm, sbuf, sem):
        pltpu.async_copy(mm, agg.at[sbuf.at[0]], sem, add=True)

    def drain_scatter(mm, sbuf, sem):
        pltpu.make_async_copy(mm, agg.at[sbuf.at[0]], sem).wait()

    def compute(gbuf, sbuf, pa, pb, cc, mm):
        cnp = c * NPAD
        for j in range(B // 16):
            sbuf[0, pl.ds(j * 16, 16)] = gbuf[0, pl.ds(j * 16, 16)] - cnp

        def edge(i, carry2):
            e0 = i * 2
            for de in range(2):
                e = e0 + de
                for j in range(2):
                    fo = j * 16
                    so = 32 + j * 16
                    f = pa[e, pl.ds(fo, 16)] + pb[e, pl.ds(fo, 16)] + cc[e, pl.ds(fo, 16)]
                    g = pa[e, pl.ds(so, 16)] + pb[e, pl.ds(so, 16)] + cc[e, pl.ds(so, 16)]
                    sig = 1.0 / (1.0 + jnp.exp(-f))
                    t = jnp.exp(-jnp.abs(g))
                    lp = t * (0.99979234 + t * (-0.49697742 + t * (0.31458911 + t * (-0.18878073 + t * (0.08172558 + t * -0.017207785)))))
                    sp = jnp.maximum(g, 0.0) + lp
                    mm[e, pl.ds(fo, 16)] = sig * sp
            return carry2

        lax.fori_loop(0, B // 2, edge, 0)

    fire_load(cbase, gA, ccA, semLA)
    fire_load(cbase + 1, gB, ccB, semLB)
    drain_load(gA, ccA, semLA)
    fire_gather(gA, paA, pbA, semGA)

    def body(g, carry):
        m_a = cbase + 2 * g
        drain_gather(gA, paA, pbA, semGA)
        drain_load(gB, ccB, semLB)
        fire_gather(gB, paB, pbB, semGB)

        @pl.when(g > 0)
        def _():
            drain_scatter(mmA, sA, semSA)

        compute(gA, sA, paA, pbA, ccA, mmA)
        fire_scatter(mmA, sA, semSA)

        @pl.when(g < NITER - 1)
        def _():
            fire_load(m_a + 2, gA, ccA, semLA)

        drain_gather(gB, paB, pbB, semGB)

        @pl.when(g > 0)
        def _():
            drain_scatter(mmB, sB, semSB)

        compute(gB, sB, paB, pbB, ccB, mmB)
        fire_scatter(mmB, sB, semSB)

        @pl.when(g < NITER - 1)
        def _():
            fire_load(m_a + 3, gB, ccB, semLB)
            drain_load(gA, ccA, semLA)
            fire_gather(gA, paA, pbA, semGA)

        return carry

    lax.fori_loop(0, NITER, body, 0)
    drain_scatter(mmA, sA, semSA)
    drain_scatter(mmB, sB, semSB)
    plsc.subcore_barrier()
    pltpu.sync_copy(agg.at[pl.ds(s * RPT, RPT)],
                    out_hbm.at[pl.ds(c * NPAD + s * RPT, RPT)])


def _proj_write(xx, wa_ref, wb_ref, ts_ref, td_ref):
    ta = jnp.dot(xx, wa_ref[...], preferred_element_type=F32)
    ts_ref[0] = ta[:, :64]
    ts_ref[1] = ta[:, 64:]
    tb = jnp.dot(xx, wb_ref[...], preferred_element_type=F32)
    td_ref[0] = tb[:, :64]
    td_ref[1] = tb[:, 64:]


def _fc1_body(x_ref, wfc_ref, bfc_ref, wa_ref, wb_ref, xx_ref, ts_ref, td_ref):
    xx = jnp.maximum(jnp.dot(x_ref[...], wfc_ref[...],
                             preferred_element_type=F32) + bfc_ref[...], 0.0)
    xx_ref[...] = xx
    _proj_write(xx, wa_ref, wb_ref, ts_ref, td_ref)


def _upd_body(xp_ref, a0_ref, a1_ref, inv_ref, wa_ref, wb_ref,
              y_ref, ts_ref, td_ref):
    agg = jnp.concatenate([a0_ref[...], a1_ref[...]], axis=1)
    y = jnp.maximum(xp_ref[...] + agg * inv_ref[...], 0.0)
    y_ref[...] = y
    _proj_write(y, wa_ref, wb_ref, ts_ref, td_ref)


def _gru_body(xp_ref, a0_ref, a1_ref, inv_ref, h_ref, wih_ref, whh_ref,
              bih_ref, bhh_ref, wa_ref, wb_ref, h_out_ref, ts_ref, td_ref):
    agg = jnp.concatenate([a0_ref[...], a1_ref[...]], axis=1)
    m = jnp.maximum(xp_ref[...] + agg * inv_ref[...], 0.0)
    h = h_ref[...]
    gi = jnp.dot(m, wih_ref[...], preferred_element_type=F32) + bih_ref[...]
    gh = jnp.dot(h, whh_ref[...], preferred_element_type=F32) + bhh_ref[...]
    r = jax.nn.sigmoid(gi[:, :64] + gh[:, :64])
    z = jax.nn.sigmoid(gi[:, 64:128] + gh[:, 64:128])
    n = jnp.tanh(gi[:, 128:] + r * gh[:, 128:])
    hn = (1.0 - z) * n + z * h
    h_out_ref[...] = hn
    _proj_write(hn, wa_ref, wb_ref, ts_ref, td_ref)


def _econst_body(ea_ref, wfe_ref, wse_ref, bf_ref, bs_ref, out_ref):
    cf = jnp.dot(ea_ref[...], wfe_ref[...], preferred_element_type=F32) + bf_ref[...]
    cs = jnp.dot(ea_ref[...], wse_ref[...], preferred_element_type=F32) + bs_ref[...]
    out_ref[0] = jnp.concatenate([cf[:, :32], cs[:, :32]], axis=1)
    out_ref[1] = jnp.concatenate([cf[:, 32:], cs[:, 32:]], axis=1)


def _readout_body(xc_ref, wsh_ref, bsh_ref, w2c_ref, b2c_ref, w3c_ref,
                  b3c_ref, w2d_ref, b2d_ref, w3d_ref, b3d_ref, oc_ref, od_ref):
    h = jnp.maximum(jnp.dot(xc_ref[...], wsh_ref[...],
                            preferred_element_type=F32) + bsh_ref[...], 0.0)
    a = jnp.maximum(jnp.dot(h, w2c_ref[...],
                            preferred_element_type=F32) + b2c_ref[...], 0.0)
    oc_ref[...] = jax.nn.sigmoid(
        jnp.dot(a, w3c_ref[...], preferred_element_type=F32) + b3c_ref[...])
    d = jnp.maximum(jnp.dot(h, w2d_ref[...],
                            preferred_element_type=F32) + b2d_ref[...], 0.0)
    od_ref[...] = jnp.dot(d, w3d_ref[...], preferred_element_type=F32) + b3d_ref[...]


_GRID = (NPAD // BLK,)


def _full(shape):
    return pl.BlockSpec(shape, lambda i: tuple(0 for _ in shape))


_NODE_OUT = [
    jax.ShapeDtypeStruct((NPAD, 64), F32),
    jax.ShapeDtypeStruct((2, NPAD, 64), F32),
    jax.ShapeDtypeStruct((2, NPAD, 64), F32),
]
_NODE_OUT_SPECS = [
    pl.BlockSpec((BLK, 64), lambda i: (i, 0)),
    pl.BlockSpec((2, BLK, 64), lambda i: (0, i, 0)),
    pl.BlockSpec((2, BLK, 64), lambda i: (0, i, 0)),
]

_fc1_proj = pl.pallas_call(
    _fc1_body,
    grid=_GRID,
    in_specs=[
        pl.BlockSpec((BLK, 9), lambda i: (i, 0)),
        _full((9, 64)), _full((1, 64)), _full((64, 128)), _full((64, 128)),
    ],
    out_specs=_NODE_OUT_SPECS,
    out_shape=_NODE_OUT,
)

_upd_proj = pl.pallas_call(
    _upd_body,
    grid=_GRID,
    in_specs=[
        pl.BlockSpec((BLK, 64), lambda i: (i, 0)),
        pl.BlockSpec((BLK, 32), lambda i: (i, 0)),
        pl.BlockSpec((BLK, 32), lambda i: (i, 0)),
        pl.BlockSpec((BLK, 1), lambda i: (i, 0)),
        _full((64, 128)), _full((64, 128)),
    ],
    out_specs=_NODE_OUT_SPECS,
    out_shape=_NODE_OUT,
)

_gru_proj = pl.pallas_call(
    _gru_body,
    grid=_GRID,
    in_specs=[
        pl.BlockSpec((BLK, 64), lambda i: (i, 0)),
        pl.BlockSpec((BLK, 32), lambda i: (i, 0)),
        pl.BlockSpec((BLK, 32), lambda i: (i, 0)),
        pl.BlockSpec((BLK, 1), lambda i: (i, 0)),
        pl.BlockSpec((BLK, 64), lambda i: (i, 0)),
        _full((64, 192)), _full((64, 192)), _full((1, 192)), _full((1, 192)),
        _full((64, 128)), _full((64, 128)),
    ],
    out_specs=_NODE_OUT_SPECS,
    out_shape=_NODE_OUT,
)

_edge_const = pl.pallas_call(
    _econst_body,
    grid=(EPAD // BLK,),
    in_specs=[
        pl.BlockSpec((BLK, 3), lambda i: (i, 0)),
        _full((3, 64)), _full((3, 64)), _full((1, 64)), _full((1, 64)),
    ],
    out_specs=[pl.BlockSpec((2, BLK, 64), lambda i: (0, i, 0))],
    out_shape=[jax.ShapeDtypeStruct((2, EPAD, 64), F32)],
)

_readout = pl.pallas_call(
    _readout_body,
    out_shape=[jax.ShapeDtypeStruct((G, 2), F32),
               jax.ShapeDtypeStruct((G, 2), F32)],
)


def kernel(x, edge_index_1, weight_1, edge_index_2, weight_2, batch,
           W_fc1, b_fc1, Wf, bf, Ws, bs, Wih, Whh, bih, bhh,
           W_sh, b_sh, W2c, b2c, W3c, b3c, W2d, b2d, W3d, b3d):
    WfA, WfB, WfE = Wf[:64], Wf[64:128], Wf[128:]
    WsA, WsB, WsE = Ws[:64], Ws[64:128], Ws[128:]
    WA = jnp.concatenate([WfA[:, :32], WsA[:, :32], WfA[:, 32:], WsA[:, 32:]], axis=1)
    WB = jnp.concatenate([WfB[:, :32], WsB[:, :32], WfB[:, 32:], WsB[:, 32:]], axis=1)
    bf2 = bf.reshape(1, 64)
    bs2 = bs.reshape(1, 64)

    xp = jnp.zeros((NPAD, 9), F32).at[:N].set(x)
    zer = jnp.zeros((RPT, 32), F32)

    def prep_edges(ei, ea):
        src = ei[0].astype(jnp.int32)
        dst = ei[1].astype(jnp.int32)
        padv = jnp.full((EPAD - E,), N, jnp.int32)
        srcp = jnp.concatenate([src, padv])
        dstp = jnp.concatenate([dst, padv])
        gs2 = jnp.concatenate([srcp, srcp + NPAD]).reshape(2, NCHK, B)
        gd2 = jnp.concatenate([dstp, dstp + NPAD]).reshape(2, NCHK, B)
        idxr = jnp.stack([gs2, gd2], axis=2).reshape(2 * NCHK * 2, B)
        cnt = jnp.bincount(src, length=NPAD).astype(F32)
        inv = (1.0 / jnp.maximum(cnt, 1.0)).reshape(NPAD, 1)
        eap = jnp.zeros((EPAD, 3), F32).at[:E].set(ea)
        (ce,) = _edge_const(eap, WfE, WsE, bf2, bs2)
        return idxr, inv, ce.reshape(2 * EPAD, 64)

    idx1, inv1, C1 = prep_edges(edge_index_1, weight_1)
    idx2, inv2, C2 = prep_edges(edge_index_2, weight_2)

    xx, ts, td = _fc1_proj(xp, W_fc1, b_fc1.reshape(1, 64), WA, WB)
    h = xx
    WihT = Wih.T
    WhhT = Whh.T
    bih2 = bih.reshape(1, 192)
    bhh2 = bhh.reshape(1, 192)

    for _ in range(3):
        agg = _sc_edge(ts.reshape(2 * NPAD, 64), td.reshape(2 * NPAD, 64),
                       C2, idx2, zer).reshape(2, NPAD, 32)
        y, ts, td = _upd_proj(xx, agg[0], agg[1], inv2, WA, WB)
        agg = _sc_edge(ts.reshape(2 * NPAD, 64), td.reshape(2 * NPAD, 64),
                       C1, idx1, zer).reshape(2, NPAD, 32)
        h, ts, td = _gru_proj(y, agg[0], agg[1], inv1, h, WihT, WhhT,
                              bih2, bhh2, WA, WB)
        xx = h

    uniq = jnp.unique(batch, size=G)
    idx_ct = jnp.searchsorted(batch, uniq)
    xc = xx[idx_ct]
    oc, od = _readout(xc, W_sh, b_sh.reshape(1, 64), W2c, b2c.reshape(1, 64),
                      W3c, b3c.reshape(1, 2), W2d, b2d.reshape(1, 64),
                      W3d, b3d.reshape(1, 2))
    return (oc, od)
